# baseline (speedup 1.0000x reference)
import functools
import jax
import jax.numpy as jnp
import numpy as np
from jax.experimental import pallas as pl
from jax.experimental.pallas import tpu as pltpu

_B = 4
_P0 = 4096
_K = 64
_BIG = 1e10
_BIGI = int(np.float32(_BIG).view(np.int32))
_EPS = 1e-5
_F32 = jnp.float32
_I32 = jnp.int32


def _dot(a, b):
    return jnp.dot(a, b, preferred_element_type=_F32,
                   precision=jax.lax.Precision.HIGHEST)


def _fps_body(px_ref, py_ref, pz_ref, o_ref, *, S, P):
    B, RR, _ = px_ref.shape
    px, py, pz = px_ref[...], py_ref[...], pz_ref[...]
    row = jax.lax.broadcasted_iota(_I32, (B, RR, 128), 1)
    col = jax.lax.broadcasted_iota(_I32, (B, RR, 128), 2)
    flat = row * 128 + col
    pad = flat < P
    for b in range(B):
        o_ref[b, 0] = jnp.int32(0)

    dmin0 = tuple(jnp.where(pad[b], jnp.inf, -jnp.inf).astype(_F32)
                  for b in range(B))
    last0 = tuple(jnp.int32(0) for _ in range(B))

    def body(i, carry):
        dmins, lasts = carry
        new_d = []
        new_l = []
        for b in range(B):
            sel = flat[b] == lasts[b]
            xl = jnp.sum(jnp.where(sel, px[b], 0.0))
            yl = jnp.sum(jnp.where(sel, py[b], 0.0))
            zl = jnp.sum(jnp.where(sel, pz[b], 0.0))
            dx = px[b] - xl
            dy = py[b] - yl
            dz = pz[b] - zl
            d = dx * dx + dy * dy + dz * dz
            dm = jnp.minimum(dmins[b], jnp.where(pad[b], d, -jnp.inf))
            m = jnp.max(dm)
            nxt = jnp.min(jnp.where(dm == m, flat[b], P))
            o_ref[b, i] = nxt
            new_d.append(dm)
            new_l.append(nxt)
        return tuple(new_d), tuple(new_l)

    jax.lax.fori_loop(1, S, body, (dmin0, last0))


def _fps(pos, S):
    B, P, _ = pos.shape
    RR = max(P // 128, 8)
    PP = RR * 128
    planes = []
    for c in range(3):
        pc = pos[..., c]
        if PP > P:
            pc = jnp.pad(pc, ((0, 0), (0, PP - P)))
        planes.append(pc.reshape(B, RR, 128))
    return pl.pallas_call(
        functools.partial(_fps_body, S=S, P=P),
        out_shape=jax.ShapeDtypeStruct((B, S), _I32),
        out_specs=pl.BlockSpec(memory_space=pltpu.SMEM),
    )(*planes)


def _sel_body(cx_ref, cy_ref, cz_ref, px_ref, py_ref, pz_ref,
              mask_ref, scnt_ref, *, r2):
    cx, cy, cz = cx_ref[0], cy_ref[0], cz_ref[0]
    px, py, pz = px_ref[0], py_ref[0], pz_ref[0]
    dx = cx - px
    dy = cy - py
    dz = cz - pz
    d2 = dx * dx + dy * dy + dz * dz
    inr = d2 <= r2
    d2i = jax.lax.bitcast_convert_type(
        jnp.where(inr, d2, _BIG).astype(_F32), _I32)
    cnt = jnp.sum(inr.astype(_I32), axis=1, keepdims=True)
    kt = jnp.minimum(cnt, _K)

    def bs(_, lohi):
        lo, hi = lohi
        mid = lo + (hi - lo) // 2
        c = jnp.sum((d2i <= mid).astype(_I32), axis=1, keepdims=True)
        ge = c >= kt
        return jnp.where(ge, lo, mid), jnp.where(ge, mid, hi)

    lo0 = jnp.full_like(cnt, -1)
    hi0 = jnp.full_like(cnt, _BIGI)
    _, hi = jax.lax.fori_loop(0, 31, bs, (lo0, hi0))
    sel = (d2i <= hi) & inr
    mask_ref[0] = sel.astype(_I32)
    scnt_ref[0] = jnp.minimum(
        jnp.sum(sel.astype(_I32), axis=1, keepdims=True), _K)


def _select(pos_c, pos, r, Ss):
    B, S, _ = pos_c.shape
    P = pos.shape[1]
    cs = [pos_c[..., c:c + 1] for c in range(3)]
    ps = [pos[..., c].reshape(B, 1, P) for c in range(3)]
    grid = (B, S // Ss)
    c_spec = pl.BlockSpec((1, Ss, 1), lambda b, s: (b, s, 0))
    p_spec = pl.BlockSpec((1, 1, P), lambda b, s: (b, 0, 0))
    mask, scnt = pl.pallas_call(
        functools.partial(_sel_body, r2=r * r),
        grid=grid,
        in_specs=[c_spec] * 3 + [p_spec] * 3,
        out_specs=[pl.BlockSpec((1, Ss, P), lambda b, s: (b, s, 0)),
                   pl.BlockSpec((1, Ss, 1), lambda b, s: (b, s, 0))],
        out_shape=[jax.ShapeDtypeStruct((B, S, P), _I32),
                   jax.ShapeDtypeStruct((B, S, 1), _I32)],
    )(*cs, *ps)
    return mask, scnt


def _valid_mask(scnt, Ts):
    return jax.lax.broadcasted_iota(_I32, (Ts, _K, 1), 1) < scnt[:, :, None]


def _acc(ref, val):
    @pl.when(pl.program_id(0) == 0)
    def _():
        ref[...] = val

    @pl.when(pl.program_id(0) != 0)
    def _():
        ref[...] = ref[...] + val


def _e1_body(catj_ref, pcp_ref, scnt_ref, W1_ref, b1_ref,
             h1_ref, s1_ref, s2_ref):
    Ts, K, Cw = catj_ref.shape
    C1 = W1_ref.shape[1]
    msg = catj_ref[...] - pcp_ref[...][:, None, :]
    h1 = _dot(msg.reshape(Ts * K, Cw), W1_ref[...]) + b1_ref[...][0]
    h1 = h1.reshape(Ts, K, C1)
    h1_ref[...] = h1
    v = _valid_mask(scnt_ref[...], Ts)
    s1 = jnp.sum(jnp.where(v, h1, 0.0), axis=(0, 1)).reshape(1, -1)
    s2 = jnp.sum(jnp.where(v, h1 * h1, 0.0), axis=(0, 1)).reshape(1, -1)
    _acc(s1_ref, s1)
    _acc(s2_ref, s2)


def _e2_body(h1_ref, scnt_ref, mu_ref, rs_ref, ga_ref, be_ref,
             W2_ref, b2_ref, h2_ref, s1_ref, s2_ref):
    Ts, K, C1 = h1_ref.shape
    C2 = W2_ref.shape[1]
    g1 = jnp.maximum(
        ga_ref[...][0] * ((h1_ref[...] - mu_ref[...][0]) * rs_ref[...][0])
        + be_ref[...][0], 0.0)
    h2 = _dot(g1.reshape(Ts * K, C1), W2_ref[...]) + b2_ref[...][0]
    h2 = h2.reshape(Ts, K, C2)
    h2_ref[...] = h2
    v = _valid_mask(scnt_ref[...], Ts)
    s1 = jnp.sum(jnp.where(v, h2, 0.0), axis=(0, 1)).reshape(1, -1)
    s2 = jnp.sum(jnp.where(v, h2 * h2, 0.0), axis=(0, 1)).reshape(1, -1)
    _acc(s1_ref, s1)
    _acc(s2_ref, s2)


def _e3_body(h2_ref, scnt_ref, mu_ref, rs_ref, ga_ref, be_ref, o_ref):
    Ts = h2_ref.shape[0]
    g2 = jnp.maximum(
        ga_ref[...][0] * ((h2_ref[...] - mu_ref[...][0]) * rs_ref[...][0])
        + be_ref[...][0], 0.0)
    v = _valid_mask(scnt_ref[...], Ts)
    o_ref[...] = jnp.max(jnp.where(v, g2, -_BIG), axis=1)


def _finalize(s1, s2, cntT):
    mu = s1[0] / cntT
    var = jnp.maximum(s2[0] / cntT - mu * mu, 0.0)
    return mu.reshape(1, -1), jax.lax.rsqrt(var + _EPS).reshape(1, -1)


def _row_spec(Ts, w):
    return pl.BlockSpec((Ts, w), lambda i: (i, 0))


def _edge_spec(Ts, C):
    return pl.BlockSpec((Ts, _K, C), lambda i: (i, 0, 0))


def _const_spec(w):
    return pl.BlockSpec((1, w), lambda i: (0, 0))


def _edge_mlp(catj, pcp, scnt, l1, l2, Ts):
    E, _, Cw = catj.shape
    C1 = l1['W'].shape[1]
    C2 = l2['W'].shape[1]
    grid = (E // Ts,)
    h1, s1, s2 = pl.pallas_call(
        _e1_body, grid=grid,
        in_specs=[_edge_spec(Ts, Cw), _row_spec(Ts, Cw), _row_spec(Ts, 1),
                  pl.BlockSpec((Cw, C1), lambda i: (0, 0)), _const_spec(C1)],
        out_specs=[_edge_spec(Ts, C1),
                   pl.BlockSpec((1, C1), lambda i: (0, 0)),
                   pl.BlockSpec((1, C1), lambda i: (0, 0))],
        out_shape=[jax.ShapeDtypeStruct((E, _K, C1), _F32)]
        + [jax.ShapeDtypeStruct((1, C1), _F32)] * 2,
    )(catj, pcp, scnt, l1['W'], l1['b'].reshape(1, C1))
    cntT = jnp.maximum(jnp.sum(scnt).astype(_F32), 1.0)
    mu1, rs1 = _finalize(s1, s2, cntT)

    h2, s1b, s2b = pl.pallas_call(
        _e2_body, grid=grid,
        in_specs=[_edge_spec(Ts, C1), _row_spec(Ts, 1)]
        + [_const_spec(C1)] * 4
        + [pl.BlockSpec((C1, C2), lambda i: (0, 0)), _const_spec(C2)],
        out_specs=[_edge_spec(Ts, C2)]
        + [pl.BlockSpec((1, C2), lambda i: (0, 0))] * 2,
        out_shape=[jax.ShapeDtypeStruct((E, _K, C2), _F32)]
        + [jax.ShapeDtypeStruct((1, C2), _F32)] * 2,
    )(h1, scnt, mu1, rs1, l1['gamma'].reshape(1, C1),
      l1['beta'].reshape(1, C1), l2['W'], l2['b'].reshape(1, C2))
    mu2, rs2 = _finalize(s1b, s2b, cntT)

    out = pl.pallas_call(
        _e3_body, grid=grid,
        in_specs=[_edge_spec(Ts, C2), _row_spec(Ts, 1)]
        + [_const_spec(C2)] * 4,
        out_specs=_row_spec(Ts, C2),
        out_shape=jax.ShapeDtypeStruct((E, C2), _F32),
    )(h2, scnt, mu2, rs2, l2['gamma'].reshape(1, C2),
      l2['beta'].reshape(1, C2))
    return out


def _sa_level(x, pos, S, r, layers, Ss, Ts):
    B, P, Cin = x.shape
    idx = _fps(pos, S)
    pos_c = jnp.take_along_axis(pos, idx[..., None].astype(_I32), axis=1)
    mask, scnt = _select(pos_c, pos, r, Ss)

    rank = jnp.cumsum(mask, axis=-1) - 1
    iota = jnp.arange(P, dtype=_I32)
    ok = (mask > 0) & (rank < _K)
    col = jnp.where(ok, rank, _K)
    bb = jnp.arange(B, dtype=_I32)[:, None, None]
    ss = jnp.arange(S, dtype=_I32)[None, :, None]
    nbr = jnp.zeros((B, S, _K + 1), _I32).at[
        bb + 0 * col, ss + 0 * col, col].set(
        jnp.broadcast_to(iota[None, None, :], col.shape))[..., :_K]

    l1, l2 = layers
    cat = jnp.concatenate([x, pos], axis=-1).reshape(B * P, Cin + 3)
    pcp = jnp.concatenate(
        [jnp.zeros((B, S, Cin), _F32), pos_c], axis=-1).reshape(B * S, -1)
    gidx = (nbr + (jnp.arange(B, dtype=_I32) * P)[:, None, None]).reshape(-1)
    catj = cat[gidx].reshape(B * S, _K, -1)

    out = _edge_mlp(catj, pcp, scnt.reshape(B * S, 1), l1, l2, Ts)
    return out.reshape(B, S, -1), pos_c


def _knn3_body(tx_ref, ty_ref, tz_ref, sx_ref, sy_ref, sz_ref,
               i_ref, w_ref, *, Ps):
    tx, ty, tz = tx_ref[0], ty_ref[0], tz_ref[0]
    sx, sy, sz = sx_ref[0], sy_ref[0], sz_ref[0]
    dx = tx - sx
    dy = ty - sy
    dz = tz - sz
    d2 = dx * dx + dy * dy + dz * dz
    iota = jax.lax.broadcasted_iota(_I32, d2.shape, 1)
    d2w = d2
    for j in range(3):
        mn = jnp.min(d2w, axis=1, keepdims=True)
        sel = jnp.min(jnp.where(d2w == mn, iota, Ps), axis=1, keepdims=True)
        i_ref[0, :, j:j + 1] = sel
        w_ref[0, :, j:j + 1] = 1.0 / jnp.maximum(mn, 1e-16)
        d2w = jnp.where(iota == sel, _BIG, d2w)


def _knn3(pos_src, pos_tgt, Tp):
    B, Ps, _ = pos_src.shape
    Pt = pos_tgt.shape[1]
    ts = [pos_tgt[..., c:c + 1] for c in range(3)]
    ss = [pos_src[..., c].reshape(B, 1, Ps) for c in range(3)]
    grid = (B, Pt // Tp)
    return pl.pallas_call(
        functools.partial(_knn3_body, Ps=Ps),
        grid=grid,
        in_specs=[pl.BlockSpec((1, Tp, 1), lambda b, p: (b, p, 0))] * 3
        + [pl.BlockSpec((1, 1, Ps), lambda b, p: (b, 0, 0))] * 3,
        out_specs=[pl.BlockSpec((1, Tp, 3), lambda b, p: (b, p, 0))] * 2,
        out_shape=[jax.ShapeDtypeStruct((B, Pt, 3), _I32),
                   jax.ShapeDtypeStruct((B, Pt, 3), _F32)],
    )(*ts, *ss)


def _rows_acc(a, s1_ref, s2_ref):
    s1 = jnp.sum(a, axis=0, keepdims=True)
    s2 = jnp.sum(a * a, axis=0, keepdims=True)
    _acc(s1_ref, s1)
    _acc(s2_ref, s2)


def _interp_lin_body(xs_ref, w3_ref, xk_ref, W1_ref, b1_ref,
                     a_ref, s1_ref, s2_ref, *, C):
    xs = xs_ref[...]
    w3 = w3_ref[...]
    w0 = w3[:, 0:1]
    w1 = w3[:, 1:2]
    w2 = w3[:, 2:3]
    num = (w0 * xs[:, :C] + w1 * xs[:, C:2 * C]) + w2 * xs[:, 2 * C:]
    xi = num / ((w0 + w1) + w2)
    h = jnp.concatenate([xi, xk_ref[...]], axis=-1)
    a = _dot(h, W1_ref[...]) + b1_ref[...]
    a_ref[...] = a
    _rows_acc(a, s1_ref, s2_ref)


def _bnrelu_lin_body(a_ref, mu_ref, rs_ref, ga_ref, be_ref, W_ref, b_ref,
                     o_ref, s1_ref, s2_ref):
    g = jnp.maximum(
        ga_ref[...] * ((a_ref[...] - mu_ref[...]) * rs_ref[...])
        + be_ref[...], 0.0)
    o = _dot(g, W_ref[...]) + b_ref[...]
    o_ref[...] = o
    _rows_acc(o, s1_ref, s2_ref)


def _bnrelu_body(a_ref, mu_ref, rs_ref, ga_ref, be_ref, o_ref):
    o_ref[...] = jnp.maximum(
        ga_ref[...] * ((a_ref[...] - mu_ref[...]) * rs_ref[...])
        + be_ref[...], 0.0)


def _lin_body(a_ref, mu_ref, rs_ref, ga_ref, be_ref, W_ref, b_ref, o_ref):
    g = jnp.maximum(
        ga_ref[...] * ((a_ref[...] - mu_ref[...]) * rs_ref[...])
        + be_ref[...], 0.0)
    o_ref[...] = _dot(g, W_ref[...]) + b_ref[...]


def _fp_level(x_src, pos_src, x_skip, pos_tgt, layers, Tp, Tr):
    B, Ps, C = x_src.shape
    Pt = pos_tgt.shape[1]
    idx3, w3 = _knn3(pos_src, pos_tgt, Tp)
    xs = jnp.take_along_axis(
        x_src[:, :, None, :], idx3[..., None], axis=1)
    R = B * Pt
    xs = xs.reshape(R, 3 * C)
    w3 = w3.reshape(R, 3)
    xk = x_skip.reshape(R, -1)
    Ck = xk.shape[1]
    l1, l2 = layers
    C1, C2 = l1['W'].shape[1], l2['W'].shape[1]
    grid = (R // Tr,)
    sspec = [pl.BlockSpec((1, C1), lambda i: (0, 0))] * 2
    a1, s1, s2 = pl.pallas_call(
        functools.partial(_interp_lin_body, C=C), grid=grid,
        in_specs=[_row_spec(Tr, 3 * C), _row_spec(Tr, 3), _row_spec(Tr, Ck),
                  pl.BlockSpec((C + Ck, C1), lambda i: (0, 0)),
                  _const_spec(C1)],
        out_specs=[_row_spec(Tr, C1)] + sspec,
        out_shape=[jax.ShapeDtypeStruct((R, C1), _F32)]
        + [jax.ShapeDtypeStruct((1, C1), _F32)] * 2,
    )(xs, w3, xk, l1['W'], l1['b'].reshape(1, C1))
    mu1, rs1 = _finalize(s1, s2, float(R))

    sspec2 = [pl.BlockSpec((1, C2), lambda i: (0, 0))] * 2
    a2, s1b, s2b = pl.pallas_call(
        _bnrelu_lin_body, grid=grid,
        in_specs=[_row_spec(Tr, C1)] + [_const_spec(C1)] * 4
        + [pl.BlockSpec((C1, C2), lambda i: (0, 0)), _const_spec(C2)],
        out_specs=[_row_spec(Tr, C2)] + sspec2,
        out_shape=[jax.ShapeDtypeStruct((R, C2), _F32)]
        + [jax.ShapeDtypeStruct((1, C2), _F32)] * 2,
    )(a1, mu1, rs1, l1['gamma'].reshape(1, C1), l1['beta'].reshape(1, C1),
      l2['W'], l2['b'].reshape(1, C2))
    mu2, rs2 = _finalize(s1b, s2b, float(R))

    out = pl.pallas_call(
        _bnrelu_body, grid=grid,
        in_specs=[_row_spec(Tr, C2)] + [_const_spec(C2)] * 4,
        out_specs=_row_spec(Tr, C2),
        out_shape=jax.ShapeDtypeStruct((R, C2), _F32),
    )(a2, mu2, rs2, l2['gamma'].reshape(1, C2), l2['beta'].reshape(1, C2))
    return out.reshape(B, Pt, C2)


def _lin_stats_body(h_ref, W_ref, b_ref, a_ref, s1_ref, s2_ref):
    a = _dot(h_ref[...], W_ref[...]) + b_ref[...]
    a_ref[...] = a
    _rows_acc(a, s1_ref, s2_ref)


def _head(h, lin1, bn, lin2, Tr=1024):
    R, Ci = h.shape
    C1 = lin1['W'].shape[1]
    C2 = lin2['W'].shape[1]
    grid = (R // Tr,)
    a1, s1, s2 = pl.pallas_call(
        _lin_stats_body, grid=grid,
        in_specs=[_row_spec(Tr, Ci),
                  pl.BlockSpec((Ci, C1), lambda i: (0, 0)),
                  _const_spec(C1)],
        out_specs=[_row_spec(Tr, C1)]
        + [pl.BlockSpec((1, C1), lambda i: (0, 0))] * 2,
        out_shape=[jax.ShapeDtypeStruct((R, C1), _F32)]
        + [jax.ShapeDtypeStruct((1, C1), _F32)] * 2,
    )(h, lin1['W'], lin1['b'].reshape(1, C1))
    mu, rs = _finalize(s1, s2, float(R))
    return pl.pallas_call(
        _lin_body, grid=grid,
        in_specs=[_row_spec(Tr, C1)] + [_const_spec(C1)] * 4
        + [pl.BlockSpec((C1, C2), lambda i: (0, 0)), _const_spec(C2)],
        out_specs=_row_spec(Tr, C2),
        out_shape=jax.ShapeDtypeStruct((R, C2), _F32),
    )(a1, mu, rs, bn['gamma'].reshape(1, C1), bn['beta'].reshape(1, C1),
      lin2['W'], lin2['b'].reshape(1, C2))


def kernel(x, pos, batch, params):
    x0 = x.reshape(_B, _P0, -1)
    p0 = pos.reshape(_B, _P0, 3)
    x1, p1 = _sa_level(x0, p0, _P0 // 4, 0.1, params['sa1'], Ss=256, Ts=32)
    x2, p2 = _sa_level(x1, p1, _P0 // 16, 0.2, params['sa2'], Ss=256, Ts=32)
    x3, p3 = _sa_level(x2, p2, _P0 // 64, 0.4, params['sa3'], Ss=64, Ts=8)
    x4, p4 = _sa_level(x3, p3, _P0 // 256, 0.8, params['sa4'], Ss=16, Ts=8)
    d3 = _fp_level(x4, p4, x3, p3, params['fp4'], Tp=64, Tr=256)
    d2 = _fp_level(d3, p3, x2, p2, params['fp3'], Tp=256, Tr=1024)
    d1 = _fp_level(d2, p2, x1, p1, params['fp2'], Tp=1024, Tr=1024)
    out = _fp_level(d1, p1, x0, p0, params['fp1'], Tp=1024, Tr=1024)
    out = out.reshape(_B * _P0, 64)
    sem = _head(out, params['sem1'], params['sem_bn'], params['sem2'])
    ins = _head(out, params['ins1'], params['ins_bn'], params['ins2'])
    return (sem, ins)
